# Initial kernel scaffold; baseline (speedup 1.0000x reference)
import functools

import jax
import jax.numpy as jnp
from jax import lax
from jax.experimental import pallas as pl
from jax.experimental.pallas import tpu as pltpu
from jax.experimental.pallas import tpu_sc as plsc

B = 8
M = 8192
D_FEAT = 64
S = 512
N_MAX = 64
EPS = 1e-5
N = B * M

T_STAT = 2048
T_SCORE = 1024

_HI = jnp.int32(1 << 30)


def _bn_relu(y, mean_ref, var_ref, g_ref, be_ref):
    xh = (y - mean_ref[...]) / jnp.sqrt(var_ref[...] + EPS)
    return jnp.maximum(xh * g_ref[...] + be_ref[...], 0.0)


def _out_stats(W, b, G_acc, s_acc, mean_ref, var_ref):
    hi = jax.lax.Precision.HIGHEST
    mu = s_acc[...] * (1.0 / N)
    E2 = G_acc[...] * (1.0 / N)
    outer = lax.dot_general(mu, mu, (((1,), (1,)), ((), ())), precision=hi)
    cov = E2 - outer
    mean_ref[...] = jnp.dot(W, mu, precision=hi) + b
    WC = jnp.dot(W, cov, precision=hi)
    var_ref[...] = jnp.sum(WC * W, axis=1, keepdims=True)


def _p0_body(x_ref, W0_ref, b0_ref, mean0_ref, var0_ref, G_acc, s_acc):
    j = pl.program_id(0)
    nt = pl.num_programs(0)
    hi = jax.lax.Precision.HIGHEST
    x = x_ref[...]

    @pl.when(j == 0)
    def _():
        G_acc[...] = jnp.zeros_like(G_acc)
        s_acc[...] = jnp.zeros_like(s_acc)

    G_acc[...] += lax.dot_general(x, x, (((1,), (1,)), ((), ())), precision=hi)
    s_acc[...] += jnp.sum(x, axis=1, keepdims=True)

    @pl.when(j == nt - 1)
    def _():
        _out_stats(W0_ref[...], b0_ref[...], G_acc, s_acc, mean0_ref, var0_ref)


def _p1_body(x_ref, W0_ref, b0_ref, mean0_ref, var0_ref, g0_ref, be0_ref,
             W1_ref, b1_ref, mean1_ref, var1_ref, G_acc, s_acc):
    j = pl.program_id(0)
    nt = pl.num_programs(0)
    hi = jax.lax.Precision.HIGHEST
    x = x_ref[...]
    y0 = jnp.dot(W0_ref[...], x) + b0_ref[...]
    a1 = _bn_relu(y0, mean0_ref, var0_ref, g0_ref, be0_ref)

    @pl.when(j == 0)
    def _():
        G_acc[...] = jnp.zeros_like(G_acc)
        s_acc[...] = jnp.zeros_like(s_acc)

    G_acc[...] += lax.dot_general(a1, a1, (((1,), (1,)), ((), ())), precision=hi)
    s_acc[...] += jnp.sum(a1, axis=1, keepdims=True)

    @pl.when(j == nt - 1)
    def _():
        _out_stats(W1_ref[...], b1_ref[...], G_acc, s_acc, mean1_ref, var1_ref)


def _p2_body(x_ref, W0_ref, b0_ref, mean0_ref, var0_ref, g0_ref, be0_ref,
             W1_ref, b1_ref, mean1_ref, var1_ref, g1_ref, be1_ref,
             W2_ref, b2_ref, mean2_ref, var2_ref, G_acc, s_acc):
    j = pl.program_id(0)
    nt = pl.num_programs(0)
    hi = jax.lax.Precision.HIGHEST
    x = x_ref[...]
    y0 = jnp.dot(W0_ref[...], x) + b0_ref[...]
    a1 = _bn_relu(y0, mean0_ref, var0_ref, g0_ref, be0_ref)
    y1 = jnp.dot(W1_ref[...], a1) + b1_ref[...]
    a2 = _bn_relu(y1, mean1_ref, var1_ref, g1_ref, be1_ref)

    @pl.when(j == 0)
    def _():
        G_acc[...] = jnp.zeros_like(G_acc)
        s_acc[...] = jnp.zeros_like(s_acc)

    G_acc[...] += lax.dot_general(a2, a2, (((1,), (1,)), ((), ())), precision=hi)
    s_acc[...] += jnp.sum(a2, axis=1, keepdims=True)

    @pl.when(j == nt - 1)
    def _():
        _out_stats(W2_ref[...], b2_ref[...], G_acc, s_acc, mean2_ref, var2_ref)


def _p3_body(x_ref, W0_ref, b0_ref, mean0_ref, var0_ref, g0_ref, be0_ref,
             W1_ref, b1_ref, mean1_ref, var1_ref, g1_ref, be1_ref,
             W2_ref, b2_ref, mean2_ref, var2_ref, g2_ref, be2_ref,
             W3_ref, b3_ref, q_ref):
    x = x_ref[...]
    y0 = jnp.dot(W0_ref[...], x) + b0_ref[...]
    a1 = _bn_relu(y0, mean0_ref, var0_ref, g0_ref, be0_ref)
    y1 = jnp.dot(W1_ref[...], a1) + b1_ref[...]
    a2 = _bn_relu(y1, mean1_ref, var1_ref, g1_ref, be1_ref)
    y2 = jnp.dot(W2_ref[...], a2) + b2_ref[...]
    a3 = _bn_relu(y2, mean2_ref, var2_ref, g2_ref, be2_ref)
    sc = jnp.dot(W3_ref[...], a3) + b3_ref[...]
    q = 0.5 + 0.5 * jnp.tanh(sc * 0.5)
    q_ref[...] = q[None]


R_TOPK = 8


def _topk_body(q_ref, idx_ref, w_ref):
    i = pl.program_id(0)
    b = (i * R_TOPK) // S
    w_ref[...] = q_ref[...]
    iota = lax.broadcasted_iota(jnp.int32, (R_TOPK, M), 1)

    def it(k, carry):
        v = w_ref[...]
        m = jnp.max(v, axis=1, keepdims=True)
        cand = jnp.where(v == m, iota, _HI)
        sel = jnp.min(cand, axis=1, keepdims=True)
        w_ref[...] = jnp.where(iota == sel, -1.0, v)
        idx_ref[:, pl.ds(k, 1)] = sel + b * M
        return carry

    lax.fori_loop(0, N_MAX, it, 0)


def _scores_q(coordinate, W0, b0, g0, be0, W1, b1, g1, be1, W2, b2, g2, be2,
              W3, b3):
    f32 = jnp.float32
    xt = coordinate.reshape(N, 3).T
    xt = jnp.pad(xt, ((0, 5), (0, 0)))
    W0p = jnp.pad(W0, ((0, 0), (0, 5)))
    cvec = lambda v: v.reshape(-1, 1)
    b0c, g0c, be0c = cvec(b0), cvec(g0), cvec(be0)
    b1c, g1c, be1c = cvec(b1), cvec(g1), cvec(be1)
    b2c, g2c, be2c = cvec(b2), cvec(g2), cvec(be2)
    b3c = cvec(b3)

    full = lambda shape: pl.BlockSpec(shape, lambda j: (0,) * len(shape))
    xspec = lambda t: pl.BlockSpec((8, t), lambda j: (0, j))
    sd = jax.ShapeDtypeStruct

    nt = N // T_STAT
    mean0, var0 = pl.pallas_call(
        _p0_body,
        grid=(nt,),
        in_specs=[xspec(T_STAT), full((32, 8)), full((32, 1))],
        out_specs=[full((32, 1)), full((32, 1))],
        out_shape=[sd((32, 1), f32), sd((32, 1), f32)],
        scratch_shapes=[pltpu.VMEM((8, 8), f32), pltpu.VMEM((8, 1), f32)],
    )(xt, W0p, b0c)

    mean1, var1 = pl.pallas_call(
        _p1_body,
        grid=(nt,),
        in_specs=[xspec(T_STAT), full((32, 8)), full((32, 1)),
                  full((32, 1)), full((32, 1)), full((32, 1)), full((32, 1)),
                  full((64, 32)), full((64, 1))],
        out_specs=[full((64, 1)), full((64, 1))],
        out_shape=[sd((64, 1), f32), sd((64, 1), f32)],
        scratch_shapes=[pltpu.VMEM((32, 32), f32), pltpu.VMEM((32, 1), f32)],
    )(xt, W0p, b0c, mean0, var0, g0c, be0c, W1, b1c)

    mean2, var2 = pl.pallas_call(
        _p2_body,
        grid=(nt,),
        in_specs=[xspec(T_STAT), full((32, 8)), full((32, 1)),
                  full((32, 1)), full((32, 1)), full((32, 1)), full((32, 1)),
                  full((64, 32)), full((64, 1)),
                  full((64, 1)), full((64, 1)), full((64, 1)), full((64, 1)),
                  full((256, 64)), full((256, 1))],
        out_specs=[full((256, 1)), full((256, 1))],
        out_shape=[sd((256, 1), f32), sd((256, 1), f32)],
        scratch_shapes=[pltpu.VMEM((64, 64), f32), pltpu.VMEM((64, 1), f32)],
    )(xt, W0p, b0c, mean0, var0, g0c, be0c, W1, b1c, mean1, var1, g1c, be1c,
      W2, b2c)

    mpt = M // T_SCORE
    q = pl.pallas_call(
        _p3_body,
        grid=(N // T_SCORE,),
        in_specs=[xspec(T_SCORE), full((32, 8)), full((32, 1)),
                  full((32, 1)), full((32, 1)), full((32, 1)), full((32, 1)),
                  full((64, 32)), full((64, 1)),
                  full((64, 1)), full((64, 1)), full((64, 1)), full((64, 1)),
                  full((256, 64)), full((256, 1)),
                  full((256, 1)), full((256, 1)), full((256, 1)), full((256, 1)),
                  full((S, 256)), full((S, 1))],
        out_specs=pl.BlockSpec((1, S, T_SCORE),
                               lambda j: (j // mpt, 0, j % mpt)),
        out_shape=sd((B, S, M), f32),
    )(xt, W0p, b0c, mean0, var0, g0c, be0c, W1, b1c, mean1, var1, g1c, be1c,
      W2, b2c, mean2, var2, g2c, be2c, W3, b3c)
    return q


def _topk_idx(q):
    q2 = q.reshape(B * S, M)
    return pl.pallas_call(
        _topk_body,
        grid=(B * S // R_TOPK,),
        in_specs=[pl.BlockSpec((R_TOPK, M), lambda i: (i, 0))],
        out_specs=pl.BlockSpec((R_TOPK, N_MAX), lambda i: (i, 0)),
        out_shape=jax.ShapeDtypeStruct((B * S, N_MAX), jnp.int32),
        scratch_shapes=[pltpu.VMEM((R_TOPK, M), jnp.float32)],
    )(q2)


_ROWS = B * S * N_MAX
_NC, _NS = 2, 16
_NW = _NC * _NS
_RPW = _ROWS // _NW
_CH = 1024


def _gather_sc_body(feat_hbm, coord_hbm, idx_hbm, gf_hbm, gp_hbm,
                    idx_v, f_v, p_v, semf, semp):
    wid = lax.axis_index("s") * _NC + lax.axis_index("c")
    base = wid * _RPW

    def chunk(ci, carry):
        off = base + ci * _CH
        pltpu.sync_copy(idx_hbm.at[pl.ds(off, _CH)], idx_v)
        cf = pltpu.async_copy(feat_hbm.at[idx_v], f_v, semf)
        cp = pltpu.async_copy(coord_hbm.at[idx_v], p_v, semp)
        cf.wait()
        cp.wait()
        pltpu.sync_copy(f_v, gf_hbm.at[pl.ds(off, _CH)])
        pltpu.sync_copy(p_v, gp_hbm.at[pl.ds(off, _CH)])
        return carry

    lax.fori_loop(0, _RPW // _CH, chunk, 0)


_gather_sc = functools.partial(
    pl.kernel,
    out_type=(jax.ShapeDtypeStruct((_ROWS, D_FEAT), jnp.float32),
              jax.ShapeDtypeStruct((_ROWS, 4), jnp.float32)),
    mesh=plsc.VectorSubcoreMesh(core_axis_name="c", subcore_axis_name="s"),
    scratch_types=[pltpu.VMEM((_CH,), jnp.int32),
                   pltpu.VMEM((_CH, D_FEAT), jnp.float32),
                   pltpu.VMEM((_CH, 4), jnp.float32),
                   pltpu.SemaphoreType.DMA,
                   pltpu.SemaphoreType.DMA],
)(_gather_sc_body)


def kernel(coordinate, feature, W0, b0, g0, be0, W1, b1, g1, be1,
           W2, b2, g2, be2, W3, b3):
    q = _scores_q(coordinate, W0, b0, g0, be0, W1, b1, g1, be1,
                  W2, b2, g2, be2, W3, b3)
    gidx = _topk_idx(q)

    feat_flat = feature.reshape(N, D_FEAT)
    coord4 = jnp.pad(coordinate.reshape(N, 3), ((0, 0), (0, 1)))
    gf_all, gp_all = _gather_sc(feat_flat, coord4, gidx.reshape(_ROWS))

    gf = gf_all.reshape(B, S, N_MAX, D_FEAT)
    gp = gp_all.reshape(B, S, N_MAX, 4)[..., :3]
    gp32 = gp[:, :, :32, :]
    gf32 = gf[:, :, :32, :]
    sampled_points = gp[:, :, 0, :]
    sampled_feature = gf[:, :, 0, :]
    return (sampled_points, gp32, gp, sampled_feature, gf32, gf)


# speedup vs baseline: 1.8502x; 1.8502x over previous
import functools

import jax
import jax.numpy as jnp
from jax import lax
from jax.experimental import pallas as pl
from jax.experimental.pallas import tpu as pltpu
from jax.experimental.pallas import tpu_sc as plsc

B = 8
M = 8192
D_FEAT = 64
S = 512
N_MAX = 64
EPS = 1e-5
N = B * M

T_STAT = 2048
T_SCORE = 1024

_HI = 1 << 30


def _dot(a, b):
    return jnp.dot(a, b)


def _bn_relu(y, mean_ref, var_ref, g_ref, be_ref):
    xh = (y - mean_ref[...]) / jnp.sqrt(var_ref[...] + EPS)
    return jnp.maximum(xh * g_ref[...] + be_ref[...], 0.0)


def _acc_stats(j, nt, y, s_acc, ss_acc, mean_ref, var_ref):
    @pl.when(j == 0)
    def _():
        s_acc[...] = jnp.zeros_like(s_acc)
        ss_acc[...] = jnp.zeros_like(ss_acc)

    s_acc[...] += jnp.sum(y, axis=1, keepdims=True)
    ss_acc[...] += jnp.sum(y * y, axis=1, keepdims=True)

    @pl.when(j == nt - 1)
    def _():
        mean = s_acc[...] * (1.0 / N)
        mean_ref[...] = mean
        var_ref[...] = ss_acc[...] * (1.0 / N) - mean * mean


def _p0_body(x_ref, W0_ref, b0_ref, mean0_ref, var0_ref, s_acc, ss_acc):
    j = pl.program_id(0)
    nt = pl.num_programs(0)
    y0 = _dot(W0_ref[...], x_ref[...]) + b0_ref[...]
    _acc_stats(j, nt, y0, s_acc, ss_acc, mean0_ref, var0_ref)


def _p1_body(x_ref, W0_ref, b0_ref, mean0_ref, var0_ref, g0_ref, be0_ref,
             W1_ref, b1_ref, mean1_ref, var1_ref, s_acc, ss_acc):
    j = pl.program_id(0)
    nt = pl.num_programs(0)
    x = x_ref[...]
    y0 = _dot(W0_ref[...], x) + b0_ref[...]
    a1 = _bn_relu(y0, mean0_ref, var0_ref, g0_ref, be0_ref)
    y1 = _dot(W1_ref[...], a1) + b1_ref[...]
    _acc_stats(j, nt, y1, s_acc, ss_acc, mean1_ref, var1_ref)


def _p2_body(x_ref, W0_ref, b0_ref, mean0_ref, var0_ref, g0_ref, be0_ref,
             W1_ref, b1_ref, mean1_ref, var1_ref, g1_ref, be1_ref,
             W2_ref, b2_ref, mean2_ref, var2_ref, s_acc, ss_acc):
    j = pl.program_id(0)
    nt = pl.num_programs(0)
    x = x_ref[...]
    y0 = _dot(W0_ref[...], x) + b0_ref[...]
    a1 = _bn_relu(y0, mean0_ref, var0_ref, g0_ref, be0_ref)
    y1 = _dot(W1_ref[...], a1) + b1_ref[...]
    a2 = _bn_relu(y1, mean1_ref, var1_ref, g1_ref, be1_ref)
    y2 = _dot(W2_ref[...], a2) + b2_ref[...]
    _acc_stats(j, nt, y2, s_acc, ss_acc, mean2_ref, var2_ref)


def _p3_body(x_ref, W0_ref, b0_ref, mean0_ref, var0_ref, g0_ref, be0_ref,
             W1_ref, b1_ref, mean1_ref, var1_ref, g1_ref, be1_ref,
             W2_ref, b2_ref, mean2_ref, var2_ref, g2_ref, be2_ref,
             W3_ref, b3_ref, q_ref):
    x = x_ref[...]
    y0 = _dot(W0_ref[...], x) + b0_ref[...]
    a1 = _bn_relu(y0, mean0_ref, var0_ref, g0_ref, be0_ref)
    y1 = _dot(W1_ref[...], a1) + b1_ref[...]
    a2 = _bn_relu(y1, mean1_ref, var1_ref, g1_ref, be1_ref)
    y2 = _dot(W2_ref[...], a2) + b2_ref[...]
    a3 = _bn_relu(y2, mean2_ref, var2_ref, g2_ref, be2_ref)
    sc = _dot(W3_ref[...], a3) + b3_ref[...]
    q = jax.nn.sigmoid(sc)
    q_ref[...] = q[None]


R_TOPK = 8


def _topk_body(q_ref, idx_ref, w_ref):
    i = pl.program_id(0)
    b = (i * R_TOPK) // S
    w_ref[...] = q_ref[...]
    iota = lax.broadcasted_iota(jnp.int32, (R_TOPK, M), 1)
    kiota = lax.broadcasted_iota(jnp.int32, (R_TOPK, N_MAX), 1)

    def it(k, res):
        v = w_ref[...]
        m = jnp.max(v, axis=1, keepdims=True)
        cand = jnp.where(v == m, iota, _HI)
        sel = jnp.min(cand, axis=1, keepdims=True)
        w_ref[...] = jnp.where(iota == sel, -1.0, v)
        return jnp.where(kiota == k, sel, res)

    res = lax.fori_loop(0, N_MAX, it,
                        jnp.zeros((R_TOPK, N_MAX), jnp.int32))
    idx_ref[...] = res + b * M


def _scores_q(coordinate, W0, b0, g0, be0, W1, b1, g1, be1, W2, b2, g2, be2,
              W3, b3):
    f32 = jnp.float32
    xt = coordinate.reshape(N, 3).T
    xt = jnp.pad(xt, ((0, 5), (0, 0)))
    W0p = jnp.pad(W0, ((0, 0), (0, 5)))
    cvec = lambda v: v.reshape(-1, 1)
    b0c, g0c, be0c = cvec(b0), cvec(g0), cvec(be0)
    b1c, g1c, be1c = cvec(b1), cvec(g1), cvec(be1)
    b2c, g2c, be2c = cvec(b2), cvec(g2), cvec(be2)
    b3c = cvec(b3)

    full = lambda shape: pl.BlockSpec(shape, lambda j: (0,) * len(shape))
    xspec = lambda t: pl.BlockSpec((8, t), lambda j: (0, j))
    sd = jax.ShapeDtypeStruct

    nt = N // T_STAT
    mean0, var0 = pl.pallas_call(
        _p0_body,
        grid=(nt,),
        in_specs=[xspec(T_STAT), full((32, 8)), full((32, 1))],
        out_specs=[full((32, 1)), full((32, 1))],
        out_shape=[sd((32, 1), f32), sd((32, 1), f32)],
        scratch_shapes=[pltpu.VMEM((32, 1), f32), pltpu.VMEM((32, 1), f32)],
    )(xt, W0p, b0c)

    mean1, var1 = pl.pallas_call(
        _p1_body,
        grid=(nt,),
        in_specs=[xspec(T_STAT), full((32, 8)), full((32, 1)),
                  full((32, 1)), full((32, 1)), full((32, 1)), full((32, 1)),
                  full((64, 32)), full((64, 1))],
        out_specs=[full((64, 1)), full((64, 1))],
        out_shape=[sd((64, 1), f32), sd((64, 1), f32)],
        scratch_shapes=[pltpu.VMEM((64, 1), f32), pltpu.VMEM((64, 1), f32)],
    )(xt, W0p, b0c, mean0, var0, g0c, be0c, W1, b1c)

    mean2, var2 = pl.pallas_call(
        _p2_body,
        grid=(nt,),
        in_specs=[xspec(T_STAT), full((32, 8)), full((32, 1)),
                  full((32, 1)), full((32, 1)), full((32, 1)), full((32, 1)),
                  full((64, 32)), full((64, 1)),
                  full((64, 1)), full((64, 1)), full((64, 1)), full((64, 1)),
                  full((256, 64)), full((256, 1))],
        out_specs=[full((256, 1)), full((256, 1))],
        out_shape=[sd((256, 1), f32), sd((256, 1), f32)],
        scratch_shapes=[pltpu.VMEM((256, 1), f32), pltpu.VMEM((256, 1), f32)],
    )(xt, W0p, b0c, mean0, var0, g0c, be0c, W1, b1c, mean1, var1, g1c, be1c,
      W2, b2c)

    mpt = M // T_SCORE
    q = pl.pallas_call(
        _p3_body,
        grid=(N // T_SCORE,),
        in_specs=[xspec(T_SCORE), full((32, 8)), full((32, 1)),
                  full((32, 1)), full((32, 1)), full((32, 1)), full((32, 1)),
                  full((64, 32)), full((64, 1)),
                  full((64, 1)), full((64, 1)), full((64, 1)), full((64, 1)),
                  full((256, 64)), full((256, 1)),
                  full((256, 1)), full((256, 1)), full((256, 1)), full((256, 1)),
                  full((S, 256)), full((S, 1))],
        out_specs=pl.BlockSpec((1, S, T_SCORE),
                               lambda j: (j // mpt, 0, j % mpt)),
        out_shape=sd((B, S, M), f32),
    )(xt, W0p, b0c, mean0, var0, g0c, be0c, W1, b1c, mean1, var1, g1c, be1c,
      W2, b2c, mean2, var2, g2c, be2c, W3, b3c)
    return q


def _topk_idx(q):
    q2 = q.reshape(B * S, M)
    return pl.pallas_call(
        _topk_body,
        grid=(B * S // R_TOPK,),
        in_specs=[pl.BlockSpec((R_TOPK, M), lambda i: (i, 0))],
        out_specs=pl.BlockSpec((R_TOPK, N_MAX), lambda i: (i, 0)),
        out_shape=jax.ShapeDtypeStruct((B * S, N_MAX), jnp.int32),
        scratch_shapes=[pltpu.VMEM((R_TOPK, M), jnp.float32)],
    )(q2)


_ROWS = B * S * N_MAX
_NC, _NS = 2, 16
_NW = _NC * _NS
_RPW = _ROWS // _NW
_CH = 512
_TW = 128


def _gather_sc_body(tab_hbm, idx_hbm, out_hbm, idx_v, row_v, sem):
    wid = lax.axis_index("s") * _NC + lax.axis_index("c")
    base = wid * _RPW

    def chunk(ci, carry):
        off = base + ci * _CH
        pltpu.sync_copy(idx_hbm.at[pl.ds(off, _CH)], idx_v)
        pltpu.async_copy(tab_hbm.at[idx_v], row_v, sem).wait()
        pltpu.sync_copy(row_v, out_hbm.at[pl.ds(off, _CH)])
        return carry

    lax.fori_loop(0, _RPW // _CH, chunk, 0)


@functools.lru_cache(maxsize=1)
def _make_gather_sc():
    return pl.kernel(
        _gather_sc_body,
        out_type=jax.ShapeDtypeStruct((_ROWS, _TW), jnp.float32),
        mesh=plsc.VectorSubcoreMesh(core_axis_name="c", subcore_axis_name="s"),
        scratch_types=[pltpu.VMEM((_CH,), jnp.int32),
                       pltpu.VMEM((_CH, _TW), jnp.float32),
                       pltpu.SemaphoreType.DMA],
    )


def _gather_sc(table, gidx_flat):
    return _make_gather_sc()(table, gidx_flat)


def kernel(coordinate, feature, W0, b0, g0, be0, W1, b1, g1, be1,
           W2, b2, g2, be2, W3, b3):
    q = _scores_q(coordinate, W0, b0, g0, be0, W1, b1, g1, be1,
                  W2, b2, g2, be2, W3, b3)
    gidx = _topk_idx(q)

    table = jnp.pad(
        jnp.concatenate([feature.reshape(N, D_FEAT),
                         coordinate.reshape(N, 3)], axis=1),
        ((0, 0), (0, _TW - D_FEAT - 3)))
    rows = _gather_sc(table, gidx.reshape(_ROWS))

    gf = rows[:, :D_FEAT].reshape(B, S, N_MAX, D_FEAT)
    gp = rows[:, D_FEAT:D_FEAT + 3].reshape(B, S, N_MAX, 3)
    gp32 = gp[:, :, :32, :]
    gf32 = gf[:, :, :32, :]
    sampled_points = gp[:, :, 0, :]
    sampled_feature = gf[:, :, 0, :]
    return (sampled_points, gp32, gp, sampled_feature, gf32, gf)
